# baseline (speedup 1.0000x reference)
import jax
import jax.numpy as jnp
from jax import lax
from jax.experimental import pallas as pl
from jax.experimental.pallas import tpu as pltpu
from jax.experimental.pallas import tpu_sc as plsc

N_ITEMS = 100000
D = 128
BATCH = 4096
HIST = 50
NC = 2
NS = 16
NW = NC * NS
CHUNK = BATCH // NW
SUB = 64
SPC = CHUNK // SUB
NQ = HIST * SPC
NBUF = 10


def _emb_body(xt_hbm, tab_hbm, out_hbm, idx_v, *rest):
    bufs = rest[:NBUF]
    gsems = rest[NBUF:2 * NBUF]
    wsems = rest[2 * NBUF:3 * NBUF]
    wid = lax.axis_index("s") * NC + lax.axis_index("c")
    col0 = wid * CHUNK

    pltpu.sync_copy(xt_hbm.at[:, pl.ds(col0, CHUNK)], idx_v)

    zeros16 = jnp.zeros((16,), jnp.float32)

    def gather(j, h, b):
        return pltpu.make_async_copy(
            tab_hbm.at[idx_v.at[j, pl.ds(h * SUB, SUB)]], bufs[b], gsems[b])

    def write(j, h, b):
        return pltpu.make_async_copy(
            bufs[b], out_hbm.at[j, pl.ds(col0 + h * SUB, SUB)], wsems[b])

    def fixup(b, j, h):
        buf = bufs[b]
        vmin = idx_v[j, pl.ds(h * SUB, 16)]
        for g in range(1, SUB // 16):
            vmin = jnp.minimum(vmin, idx_v[j, pl.ds(h * SUB + g * 16, 16)])
        m = vmin[0]
        for l in range(1, 16):
            m = jnp.minimum(m, vmin[l])

        @pl.when(m == 0)
        def _():
            def group_body(g, carry):
                iv = idx_v[j, pl.ds(h * SUB + g * 16, 16)]
                for l in range(16):
                    @pl.when(iv[l] == 0)
                    def _(l=l):
                        for cblk in range(D // 16):
                            buf[g * 16 + l, pl.ds(cblk * 16, 16)] = zeros16
                return carry

            lax.fori_loop(0, SUB // 16, group_body, 0)

    for b in range(NBUF - 2):
        gather(b // SPC, b % SPC, b).start()

    def round_body(k, carry):
        for b in range(NBUF):
            q = NBUF * k + b
            j = NBUF // SPC * k + b // SPC
            h = b % SPC
            gather(j, h, b).wait()
            fixup(b, j, h)
            write(j, h, b).start()
            nq = q + NBUF - 2
            nb = (b + NBUF - 2) % NBUF
            nj = nq // SPC
            nh = (b + NBUF - 2) % SPC

            @pl.when(nq < NQ)
            def _():
                @pl.when(q >= 2)
                def _():
                    write(nj - NBUF // SPC, nh, nb).wait()

                gather(nj, nh, nb).start()
        return carry

    lax.fori_loop(0, NQ // NBUF, round_body, 0)

    for b in range(NBUF):
        q = NQ - NBUF + b
        write(q // SPC, q % SPC, b).wait()


def kernel(X, table):
    xt = jnp.transpose(X)
    mesh = plsc.VectorSubcoreMesh(core_axis_name="c", subcore_axis_name="s")
    out = pl.kernel(
        _emb_body,
        out_type=jax.ShapeDtypeStruct((HIST, BATCH, D), jnp.float32),
        mesh=mesh,
        scratch_types=[
            pltpu.VMEM((HIST, CHUNK), jnp.int32),
            *[pltpu.VMEM((SUB, D), jnp.float32) for _ in range(NBUF)],
            *[pltpu.SemaphoreType.DMA for _ in range(2 * NBUF)],
        ],
    )(xt, table)
    return jnp.transpose(out, (1, 0, 2))


# speedup vs baseline: 1.0269x; 1.0269x over previous
import jax
import jax.numpy as jnp
from jax import lax
from jax.experimental import pallas as pl
from jax.experimental.pallas import tpu as pltpu
from jax.experimental.pallas import tpu_sc as plsc

N_ITEMS = 100000
D = 128
BATCH = 4096
HIST = 50
NC = 2
NS = 16
NW = NC * NS
CHUNK = BATCH // NW
NBUF = 5


def _emb_body(xt_hbm, tab_hbm, out_hbm, idx_v, *rest):
    bufs = rest[:NBUF]
    gsems = rest[NBUF:2 * NBUF]
    wsems = rest[2 * NBUF:3 * NBUF]
    wid = lax.axis_index("s") * NC + lax.axis_index("c")
    col0 = wid * CHUNK

    pltpu.sync_copy(xt_hbm.at[:, pl.ds(col0, CHUNK)], idx_v)

    zeros16 = jnp.zeros((16,), jnp.float32)

    def gather(j, b):
        return pltpu.make_async_copy(
            tab_hbm.at[idx_v.at[j]], bufs[b], gsems[b])

    def write(j, b):
        return pltpu.make_async_copy(
            bufs[b], out_hbm.at[j, pl.ds(col0, CHUNK)], wsems[b])

    def fixup(b, j):
        buf = bufs[b]
        vmin = idx_v[j, pl.ds(0, 16)]
        for g in range(1, CHUNK // 16):
            vmin = jnp.minimum(vmin, idx_v[j, pl.ds(g * 16, 16)])
        m = vmin[0]
        for l in range(1, 16):
            m = jnp.minimum(m, vmin[l])

        @pl.when(m == 0)
        def _():
            def group_body(g, carry):
                iv = idx_v[j, pl.ds(g * 16, 16)]
                for l in range(16):
                    @pl.when(iv[l] == 0)
                    def _(l=l):
                        for cblk in range(D // 16):
                            buf[g * 16 + l, pl.ds(cblk * 16, 16)] = zeros16
                return carry

            lax.fori_loop(0, CHUNK // 16, group_body, 0)

    for b in range(NBUF - 2):
        gather(b, b).start()

    def round_body(k, carry):
        for b in range(NBUF):
            j = NBUF * k + b
            nj = j + NBUF - 2
            nb = (b + NBUF - 2) % NBUF

            @pl.when(nj < HIST)
            def _():
                @pl.when(j >= 2)
                def _():
                    write(nj - NBUF, nb).wait()

                gather(nj, nb).start()

            gather(j, b).wait()
            fixup(b, j)
            write(j, b).start()
        return carry

    lax.fori_loop(0, HIST // NBUF, round_body, 0)

    for b in range(NBUF):
        write(HIST - NBUF + b, b).wait()


def kernel(X, table):
    xt = jnp.transpose(X)
    mesh = plsc.VectorSubcoreMesh(core_axis_name="c", subcore_axis_name="s")
    out = pl.kernel(
        _emb_body,
        out_type=jax.ShapeDtypeStruct((HIST, BATCH, D), jnp.float32),
        mesh=mesh,
        scratch_types=[
            pltpu.VMEM((HIST, CHUNK), jnp.int32),
            *[pltpu.VMEM((CHUNK, D), jnp.float32) for _ in range(NBUF)],
            *[pltpu.SemaphoreType.DMA for _ in range(2 * NBUF)],
        ],
    )(xt, table)
    return jnp.transpose(out, (1, 0, 2))


# speedup vs baseline: 1.0584x; 1.0307x over previous
import jax
import jax.numpy as jnp
from jax import lax
from jax.experimental import pallas as pl
from jax.experimental.pallas import tpu as pltpu
from jax.experimental.pallas import tpu_sc as plsc

N_ITEMS = 100000
D = 128
BATCH = 4096
HIST = 50
NC = 2
NS = 16
NW = NC * NS
CHUNK = BATCH // NW
NBUF = 5


def _emb_body(xt_hbm, tab_hbm, out_hbm, idx_v, bufs, gsems, wsems):
    wid = lax.axis_index("s") * NC + lax.axis_index("c")
    col0 = wid * CHUNK

    pltpu.sync_copy(xt_hbm.at[:, pl.ds(col0, CHUNK)], idx_v)

    zeros16 = jnp.zeros((16,), jnp.float32)

    def gather(j, b):
        return pltpu.make_async_copy(
            tab_hbm.at[idx_v.at[j]], bufs.at[b], gsems.at[b])

    def write(j, b):
        return pltpu.make_async_copy(
            bufs.at[b], out_hbm.at[j, pl.ds(col0, CHUNK)], wsems.at[b])

    def fixup(b, j):
        buf = bufs.at[b]
        vmin = idx_v[j, pl.ds(0, 16)]
        for g in range(1, CHUNK // 16):
            vmin = jnp.minimum(vmin, idx_v[j, pl.ds(g * 16, 16)])
        m = vmin[0]
        for l in range(1, 16):
            m = jnp.minimum(m, vmin[l])

        @pl.when(m == 0)
        def _():
            def group_body(g, carry):
                iv = idx_v[j, pl.ds(g * 16, 16)]
                for l in range(16):
                    @pl.when(iv[l] == 0)
                    def _(l=l):
                        for cblk in range(D // 16):
                            buf[g * 16 + l, pl.ds(cblk * 16, 16)] = zeros16
                return carry

            lax.fori_loop(0, CHUNK // 16, group_body, 0)

    for b in range(NBUF - 2):
        gather(b, b).start()

    def chunk_body(j, carry):
        b = lax.rem(j, NBUF)
        nj = j + NBUF - 2
        nb = lax.rem(nj, NBUF)

        @pl.when(nj < HIST)
        def _():
            @pl.when(j >= 2)
            def _():
                write(nj - NBUF, nb).wait()

            gather(nj, nb).start()

        gather(j, b).wait()
        fixup(b, j)
        write(j, b).start()
        return carry

    lax.fori_loop(0, HIST, chunk_body, 0)

    for b in range(NBUF):
        write(HIST - NBUF + b, b).wait()


def kernel(X, table):
    xt = jnp.transpose(X)
    mesh = plsc.VectorSubcoreMesh(core_axis_name="c", subcore_axis_name="s")
    out = pl.kernel(
        _emb_body,
        out_type=jax.ShapeDtypeStruct((HIST, BATCH, D), jnp.float32),
        mesh=mesh,
        scratch_types=[
            pltpu.VMEM((HIST, CHUNK), jnp.int32),
            pltpu.VMEM((NBUF, CHUNK, D), jnp.float32),
            pltpu.SemaphoreType.DMA((NBUF,)),
            pltpu.SemaphoreType.DMA((NBUF,)),
        ],
    )(xt, table)
    return jnp.transpose(out, (1, 0, 2))


# speedup vs baseline: 1.0603x; 1.0017x over previous
import jax
import jax.numpy as jnp
from jax import lax
from jax.experimental import pallas as pl
from jax.experimental.pallas import tpu as pltpu
from jax.experimental.pallas import tpu_sc as plsc

N_ITEMS = 100000
D = 128
BATCH = 4096
HIST = 50
NC = 2
NS = 16
NW = NC * NS
CHUNK = BATCH // NW
NBUF = 6


def _emb_body(xt_hbm, tab_hbm, out_hbm, idx_v, bufs, gsems, wsems):
    wid = lax.axis_index("s") * NC + lax.axis_index("c")
    col0 = wid * CHUNK

    pltpu.sync_copy(xt_hbm.at[:, pl.ds(col0, CHUNK)], idx_v)

    zeros16 = jnp.zeros((16,), jnp.float32)

    def gather(j, b):
        return pltpu.make_async_copy(
            tab_hbm.at[idx_v.at[j]], bufs.at[b], gsems.at[b])

    def write(j, b):
        return pltpu.make_async_copy(
            bufs.at[b], out_hbm.at[j, pl.ds(col0, CHUNK)], wsems.at[b])

    def fixup(b, j):
        buf = bufs.at[b]
        vmin = idx_v[j, pl.ds(0, 16)]
        for g in range(1, CHUNK // 16):
            vmin = jnp.minimum(vmin, idx_v[j, pl.ds(g * 16, 16)])
        m = vmin[0]
        for l in range(1, 16):
            m = jnp.minimum(m, vmin[l])

        @pl.when(m == 0)
        def _():
            def group_body(g, carry):
                iv = idx_v[j, pl.ds(g * 16, 16)]
                for l in range(16):
                    @pl.when(iv[l] == 0)
                    def _(l=l):
                        for cblk in range(D // 16):
                            buf[g * 16 + l, pl.ds(cblk * 16, 16)] = zeros16
                return carry

            lax.fori_loop(0, CHUNK // 16, group_body, 0)

    for b in range(NBUF - 2):
        gather(b, b).start()

    def chunk_body(j, carry):
        b = lax.rem(j, NBUF)
        nj = j + NBUF - 2
        nb = lax.rem(nj, NBUF)

        @pl.when(nj < HIST)
        def _():
            @pl.when(j >= 2)
            def _():
                write(nj - NBUF, nb).wait()

            gather(nj, nb).start()

        gather(j, b).wait()
        fixup(b, j)
        write(j, b).start()
        return carry

    lax.fori_loop(0, HIST, chunk_body, 0)

    for b in range(NBUF):
        j = HIST - NBUF + b
        write(j, j % NBUF).wait()


def kernel(X, table):
    xt = jnp.transpose(X)
    mesh = plsc.VectorSubcoreMesh(core_axis_name="c", subcore_axis_name="s")
    out = pl.kernel(
        _emb_body,
        out_type=jax.ShapeDtypeStruct((HIST, BATCH, D), jnp.float32),
        mesh=mesh,
        scratch_types=[
            pltpu.VMEM((HIST, CHUNK), jnp.int32),
            pltpu.VMEM((NBUF, CHUNK, D), jnp.float32),
            pltpu.SemaphoreType.DMA((NBUF,)),
            pltpu.SemaphoreType.DMA((NBUF,)),
        ],
    )(xt, table)
    return jnp.transpose(out, (1, 0, 2))


# speedup vs baseline: 1.0781x; 1.0168x over previous
import jax
import jax.numpy as jnp
from jax import lax
from jax.experimental import pallas as pl
from jax.experimental.pallas import tpu as pltpu
from jax.experimental.pallas import tpu_sc as plsc

N_ITEMS = 100000
D = 128
BATCH = 4096
HIST = 50
NC = 2
NS = 16
NW = NC * NS
CHUNK = BATCH // NW
NBUF = 7


def _emb_body(xt_hbm, tab_hbm, out_hbm, idx_v, bufs, gsems, wsems):
    wid = lax.axis_index("s") * NC + lax.axis_index("c")
    col0 = wid * CHUNK

    pltpu.sync_copy(xt_hbm.at[:, pl.ds(col0, CHUNK)], idx_v)

    zeros16 = jnp.zeros((16,), jnp.float32)

    def gather(j, b):
        return pltpu.make_async_copy(
            tab_hbm.at[idx_v.at[j]], bufs.at[b], gsems.at[b])

    def write(j, b):
        return pltpu.make_async_copy(
            bufs.at[b], out_hbm.at[j, pl.ds(col0, CHUNK)], wsems.at[b])

    def fixup(b, j):
        buf = bufs.at[b]
        vmin = idx_v[j, pl.ds(0, 16)]
        for g in range(1, CHUNK // 16):
            vmin = jnp.minimum(vmin, idx_v[j, pl.ds(g * 16, 16)])
        m = vmin[0]
        for l in range(1, 16):
            m = jnp.minimum(m, vmin[l])

        @pl.when(m == 0)
        def _():
            def group_body(g, carry):
                iv = idx_v[j, pl.ds(g * 16, 16)]
                for l in range(16):
                    @pl.when(iv[l] == 0)
                    def _(l=l):
                        for cblk in range(D // 16):
                            buf[g * 16 + l, pl.ds(cblk * 16, 16)] = zeros16
                return carry

            lax.fori_loop(0, CHUNK // 16, group_body, 0)

    for b in range(NBUF - 2):
        gather(b, b).start()

    def chunk_body(j, carry):
        b = lax.rem(j, NBUF)
        nj = j + NBUF - 2
        nb = lax.rem(nj, NBUF)

        @pl.when(nj < HIST)
        def _():
            @pl.when(j >= 2)
            def _():
                write(nj - NBUF, nb).wait()

            gather(nj, nb).start()

        gather(j, b).wait()
        fixup(b, j)
        write(j, b).start()
        return carry

    lax.fori_loop(0, HIST, chunk_body, 0)

    for b in range(NBUF):
        j = HIST - NBUF + b
        write(j, j % NBUF).wait()


def kernel(X, table):
    xt = jnp.transpose(X)
    mesh = plsc.VectorSubcoreMesh(core_axis_name="c", subcore_axis_name="s")
    out = pl.kernel(
        _emb_body,
        out_type=jax.ShapeDtypeStruct((HIST, BATCH, D), jnp.float32),
        mesh=mesh,
        scratch_types=[
            pltpu.VMEM((HIST, CHUNK), jnp.int32),
            pltpu.VMEM((NBUF, CHUNK, D), jnp.float32),
            pltpu.SemaphoreType.DMA((NBUF,)),
            pltpu.SemaphoreType.DMA((NBUF,)),
        ],
    )(xt, table)
    return jnp.transpose(out, (1, 0, 2))
